# Initial kernel scaffold; baseline (speedup 1.0000x reference)
import jax
import jax.numpy as jnp
from jax.experimental import pallas as pl


def kernel(Input):
    raise NotImplementedError("write your pallas kernel here")


# speedup vs baseline: 5.3485x; 5.3485x over previous
import jax
import jax.numpy as jnp
from jax import lax
from jax.experimental import pallas as pl
from jax.experimental.pallas import tpu as pltpu
from jax.experimental.pallas import tpu_sc as plsc

_LABEL_NUM = 4096
_LO = 0.1 / _LABEL_NUM
_HI = 0.9 + 0.1 / _LABEL_NUM

_NC = 2
_NS = 16
_NW = _NC * _NS

_ROWS = 4 * 2048
_RPW = _ROWS // _NW
_K = 16
_CHUNKS = _RPW // _K


def _body(labels_hbm, out_hbm, idx_v, buf_v):
    wid = lax.axis_index("s") * _NC + lax.axis_index("c")
    base = wid * _RPW
    pltpu.sync_copy(labels_hbm.at[pl.ds(base, _RPW)], idx_v)

    lo_vec = jnp.full((16,), _LO, jnp.float32)
    hi_vec = jnp.full((16,), _HI, jnp.float32)

    def fill(j, carry):
        buf_v[pl.ds(j * 16, 16)] = lo_vec
        return carry
    lax.fori_loop(0, _K * _LABEL_NUM // 16, fill, 0)

    row_off = lax.iota(jnp.int32, 16) * _LABEL_NUM

    def chunk(c, carry):
        cols = idx_v[pl.ds(c * _K, 16)]
        flat = row_off + cols
        plsc.store_scatter(buf_v, [flat], hi_vec)
        pltpu.sync_copy(
            buf_v, out_hbm.at[pl.ds((base + c * _K) * _LABEL_NUM, _K * _LABEL_NUM)]
        )
        plsc.store_scatter(buf_v, [flat], lo_vec)
        return carry
    lax.fori_loop(0, _CHUNKS, chunk, 0)


@jax.jit
def kernel(Input):
    B, L = Input.shape
    labels = jnp.reshape(Input, (B * L,))
    mesh = plsc.VectorSubcoreMesh(core_axis_name="c", subcore_axis_name="s")
    out = pl.kernel(
        _body,
        out_type=jax.ShapeDtypeStruct((_ROWS * _LABEL_NUM,), jnp.float32),
        mesh=mesh,
        scratch_types=[
            pltpu.VMEM((_RPW,), jnp.int32),
            pltpu.VMEM((_K * _LABEL_NUM,), jnp.float32),
        ],
        compiler_params=pltpu.CompilerParams(needs_layout_passes=False),
    )(labels)
    return jnp.reshape(out, (B, L, _LABEL_NUM))


# speedup vs baseline: 5.5544x; 1.0385x over previous
import jax
import jax.numpy as jnp
from jax import lax
from jax.experimental import pallas as pl
from jax.experimental.pallas import tpu as pltpu
from jax.experimental.pallas import tpu_sc as plsc

_LABEL_NUM = 4096
_LO = 0.1 / _LABEL_NUM
_HI = 0.9 + 0.1 / _LABEL_NUM

_NC = 2
_NS = 16
_NW = _NC * _NS

_ROWS = 4 * 2048
_RPW = _ROWS // _NW
_K = 16
_CHUNKS = _RPW // _K
_SCAT = 128


def _body(labels_hbm, out_hbm, idx_v, buf_v, flatidx_v, hi_v, fill_sem, sc_sem):
    wid = lax.axis_index("s") * _NC + lax.axis_index("c")
    base = wid * _RPW
    pltpu.sync_copy(labels_hbm.at[pl.ds(base, _RPW)], idx_v)

    lo_vec = jnp.full((16,), _LO, jnp.float32)
    hi_vec = jnp.full((16,), _HI, jnp.float32)

    def fill(j, carry):
        for u in range(16):
            buf_v[pl.ds(j * 256 + u * 16, 16)] = lo_vec
        return carry
    lax.fori_loop(0, _K * _LABEL_NUM // 256, fill, 0, unroll=False)

    copies = []
    for c in range(_CHUNKS):
        copies.append(
            pltpu.async_copy(
                buf_v,
                out_hbm.at[pl.ds((base + c * _K) * _LABEL_NUM, _K * _LABEL_NUM)],
                fill_sem,
            )
        )

    lanes = lax.iota(jnp.int32, 16)
    for c in range(_RPW // 16):
        cols = idx_v[pl.ds(c * 16, 16)]
        flat = (base + c * 16 + lanes) * _LABEL_NUM + cols
        flatidx_v[c // 8, pl.ds((c % 8) * 16, 16)] = flat
        hi_v[c // 8, pl.ds((c % 8) * 16, 16)] = hi_vec

    for cp in copies:
        cp.wait()
    scats = []
    for j in range(_RPW // _SCAT):
        scats.append(
            pltpu.async_copy(hi_v.at[j], out_hbm.at[flatidx_v.at[j]], sc_sem)
        )
    for cp in scats:
        cp.wait()


@jax.jit
def kernel(Input):
    B, L = Input.shape
    labels = jnp.reshape(Input, (B * L,))
    mesh = plsc.VectorSubcoreMesh(core_axis_name="c", subcore_axis_name="s")
    out = pl.kernel(
        _body,
        out_type=jax.ShapeDtypeStruct((_ROWS * _LABEL_NUM,), jnp.float32),
        mesh=mesh,
        scratch_types=[
            pltpu.VMEM((_RPW,), jnp.int32),
            pltpu.VMEM((_K * _LABEL_NUM,), jnp.float32),
            pltpu.VMEM((_RPW // _SCAT, _SCAT), jnp.int32),
            pltpu.VMEM((_RPW // _SCAT, _SCAT), jnp.float32),
            pltpu.SemaphoreType.DMA,
            pltpu.SemaphoreType.DMA,
        ],
        compiler_params=pltpu.CompilerParams(needs_layout_passes=False),
    )(labels)
    return jnp.reshape(out, (B, L, _LABEL_NUM))


# speedup vs baseline: 17.8174x; 3.2078x over previous
import jax
import jax.numpy as jnp
from jax import lax
from jax.experimental import pallas as pl
from jax.experimental.pallas import tpu as pltpu
from jax.experimental.pallas import tpu_sc as plsc

_B = 4
_L = 2048
_LABEL_NUM = 4096
_LO = 0.1 / _LABEL_NUM
_HI = 0.9 + 0.1 / _LABEL_NUM

_NC = 2
_NS = 16
_NW = _NC * _NS

_ROWS = _B * _L
_RPW = _ROWS // _NW
_WPB = _L // _RPW
_K = 8
_CHUNKS = _RPW // _K


def _body(labels_hbm, out_hbm, idx_v, buf0_v, buf1_v, sem0, sem1):
    wid = lax.axis_index("s") * _NC + lax.axis_index("c")
    base = wid * _RPW
    b = wid // _WPB
    l0 = (wid % _WPB) * _RPW
    pltpu.sync_copy(labels_hbm.at[pl.ds(base, _RPW)], idx_v.at[pl.ds(0, _RPW)])

    lo_vec = jnp.full((16,), _LO, jnp.float32)
    hi_vec = jnp.full((16,), _HI, jnp.float32)

    idx_v[pl.ds(_RPW, 16)] = jnp.zeros((16,), jnp.int32)

    def fill_block(buf):
        def body(j, carry):
            r = j // (_LABEL_NUM // 256)
            coff = (j % (_LABEL_NUM // 256)) * 256
            for u in range(16):
                buf[r, pl.ds(coff + u * 16, 16)] = lo_vec
            return carry
        lax.fori_loop(0, _K * _LABEL_NUM // 256, body, 0)

    fill_block(buf0_v)
    fill_block(buf1_v)

    lanes = lax.iota(jnp.int32, 16)
    rows8 = lanes & (_K - 1)
    mask8 = lanes < _K

    bufs = (buf0_v, buf1_v)
    sems = (sem0, sem1)

    def chunk_cols(c):
        return idx_v[pl.ds(c * _K, 16)]

    def start(c, p):
        cols = chunk_cols(c)
        plsc.store_scatter(bufs[p], [rows8, cols], hi_vec, mask=mask8)
        return pltpu.async_copy(
            bufs[p], out_hbm.at[b].at[pl.ds(l0 + c * _K, _K)], sems[p]
        )

    def finish(c, p, cp):
        cp.wait()
        cols = chunk_cols(c)
        plsc.store_scatter(bufs[p], [rows8, cols], lo_vec, mask=mask8)

    cp0 = start(0, 0)
    cp1 = start(1, 1)

    def loop(i, carry):
        for p in range(2):
            c = 2 * i + p
            cp = pltpu.make_async_copy(
                bufs[p], out_hbm.at[b].at[pl.ds(l0 + (c - 2) * _K, _K)], sems[p]
            )
            finish(c - 2, p, cp)
            cols = chunk_cols(c)
            plsc.store_scatter(bufs[p], [rows8, cols], hi_vec, mask=mask8)
            pltpu.async_copy(
                bufs[p], out_hbm.at[b].at[pl.ds(l0 + c * _K, _K)], sems[p]
            )
        return carry

    lax.fori_loop(1, _CHUNKS // 2, loop, 0)
    cp0.wait()
    cp1.wait()


@jax.jit
def kernel(Input):
    mesh = plsc.VectorSubcoreMesh(core_axis_name="c", subcore_axis_name="s")
    labels = jnp.reshape(Input, (_ROWS,))
    out = pl.kernel(
        _body,
        out_type=jax.ShapeDtypeStruct((_B, _L, _LABEL_NUM), jnp.float32),
        mesh=mesh,
        scratch_types=[
            pltpu.VMEM((_RPW + 16,), jnp.int32),
            pltpu.VMEM((_K, _LABEL_NUM), jnp.float32),
            pltpu.VMEM((_K, _LABEL_NUM), jnp.float32),
            pltpu.SemaphoreType.DMA,
            pltpu.SemaphoreType.DMA,
        ],
        compiler_params=pltpu.CompilerParams(needs_layout_passes=False),
    )(labels)
    return out
